# baseline (speedup 1.0000x reference)
import functools

import jax
import jax.numpy as jnp
from jax import lax
from jax.experimental import pallas as pl
from jax.experimental.pallas import tpu as pltpu
from jax.experimental.pallas import tpu_sc as plsc

N_BINS = 10
GROUP = 16
UNROLL = 8


def _sc_partials(n_rows, n_cols, n_workers):
  rows_per_worker = n_rows // n_workers
  n_groups = rows_per_worker // GROUP
  gsz = GROUP * n_cols
  acc_len = 3 * N_BINS * GROUP
  mesh = plsc.VectorSubcoreMesh(core_axis_name="c", subcore_axis_name="s")

  @functools.partial(
      pl.kernel,
      mesh=mesh,
      compiler_params=pltpu.CompilerParams(needs_layout_passes=False),
      out_type=jax.ShapeDtypeStruct((n_workers, acc_len), jnp.float32),
      scratch_types=[
          pltpu.VMEM((2 * gsz,), jnp.float32),
          pltpu.VMEM((rows_per_worker,), jnp.int32),
          pltpu.VMEM((14 * GROUP,), jnp.int32),
          pltpu.VMEM((acc_len,), jnp.float32),
          pltpu.SemaphoreType.DMA,
          pltpu.SemaphoreType.DMA,
      ],
  )
  def sc_kernel(logits_hbm, labels_hbm, out_hbm, buf, lbl_v, hist, acc,
                sem0, sem1):
    wid = lax.axis_index("s") * 2 + lax.axis_index("c")
    row0 = wid * rows_per_worker

    lane = lax.iota(jnp.int32, GROUP)
    lane_base = lane * n_cols
    lane_p32 = lane + 32
    zero_i = jnp.zeros((GROUP,), jnp.int32)
    zero_f = jnp.zeros((GROUP,), jnp.float32)
    one_i = jnp.ones((GROUP,), jnp.int32)
    sixteen_i = jnp.full((GROUP,), 16, jnp.int32)
    neg_inf = jnp.full((GROUP,), -jnp.inf, jnp.float32)

    def copy_group(j, sem):
      return pltpu.make_async_copy(
          logits_hbm.at[pl.ds((row0 + j * GROUP) * n_cols, gsz)],
          buf.at[pl.ds((j % 2) * gsz, gsz)],
          sem)

    pltpu.sync_copy(labels_hbm.at[pl.ds(row0, rows_per_worker)], lbl_v)
    for j in range(3 * N_BINS):
      acc[pl.ds(j * GROUP, GROUP)] = zero_f

    copy_group(0, sem0).start()

    def group_body(g, _):
      parity = lax.rem(g, 2)
      base_off = parity * gsz

      @pl.when(parity == 0)
      def _():
        copy_group(g, sem0).wait()

      @pl.when(parity != 0)
      def _():
        copy_group(g, sem1).wait()

      @pl.when(g + 1 < n_groups)
      def _():
        @pl.when(parity == 0)
        def _():
          copy_group(g + 1, sem1).start()

        @pl.when(parity != 0)
        def _():
          copy_group(g + 1, sem0).start()

      for b in range(14):
        hist[pl.ds(b * GROUP, GROUP)] = zero_i

      gather_base = lane_base + base_off

      def col_body(ci, carry):
        ms = list(carry[:4])
        args = list(carry[4:])
        c0 = ci * UNROLL
        for u in range(UNROLL):
          k = u % 4
          c = c0 + u
          cv = jnp.full((GROUP,), c, jnp.int32)
          x = plsc.load_gather(buf, [gather_base + cv])
          upd = x > ms[k]
          args[k] = jnp.where(upd, cv, args[k])
          ms[k] = jnp.maximum(ms[k], x)
          xc = jnp.minimum(jnp.maximum(x, -1.0), 12.0)
          t = xc.astype(jnp.int32)
          tf = t.astype(jnp.float32)
          gt = xc > tf
          a0 = lax.shift_left(t, 4) + lane_p32
          corr = jnp.where(gt, zero_i, sixteen_i)
          addr = a0 - corr
          plsc.addupdate_scatter(hist, [addr], one_i)
        return (*ms, *args)

      fin = lax.fori_loop(0, n_cols // UNROLL, col_body,
                          (neg_inf,) * 4 + (zero_i,) * 4)

      def merge(ma, aa, mb, ab):
        take_b = (mb > ma) | ((mb == ma) & (ab < aa))
        return jnp.maximum(ma, mb), jnp.where(take_b, ab, aa)

      m01, a01 = merge(fin[0], fin[4], fin[1], fin[5])
      m23, a23 = merge(fin[2], fin[6], fin[3], fin[7])
      m, arg = merge(m01, a01, m23, a23)

      lbl = lbl_v[pl.ds(g * GROUP, GROUP)]
      accv = jnp.where(arg == lbl, 1.0, 0.0).astype(jnp.float32)
      for b in range(N_BINS):
        h = hist[pl.ds((b + 2) * GROUP, GROUP)].astype(jnp.float32)
        acc[pl.ds(b * GROUP, GROUP)] += h
        acc[pl.ds((N_BINS + b) * GROUP, GROUP)] += m * h
        acc[pl.ds((2 * N_BINS + b) * GROUP, GROUP)] += accv * h
      return 0

    lax.fori_loop(0, n_groups, group_body, 0)
    pltpu.sync_copy(acc, out_hbm.at[wid])

  return sc_kernel


def _tc_finisher(total):
  def body(p_ref, out_ref):
    sums = jnp.sum(p_ref[...], axis=1, keepdims=True)
    cnt = sums[0:N_BINS, :]
    conf = sums[N_BINS:2 * N_BINS, :]
    accs = sums[2 * N_BINS:3 * N_BINS, :]
    denom = jnp.maximum(cnt, 1.0)
    prop = cnt / total
    contrib = jnp.where(cnt > 0.0,
                        jnp.abs(conf / denom - accs / denom) * prop, 0.0)
    out_ref[0, 0] = jnp.sum(contrib)

  return body


def kernel(logits, labels):
  n_rows, n_cols = logits.shape
  n_workers = 32
  flat = logits.reshape(n_rows * n_cols)
  partials = _sc_partials(n_rows, n_cols, n_workers)(flat, labels)
  pt = partials.reshape(n_workers, 3 * N_BINS, GROUP)
  pt = pt.transpose(1, 0, 2).reshape(3 * N_BINS, n_workers * GROUP)
  total = float(n_rows * n_cols)
  ece = pl.pallas_call(
      _tc_finisher(total),
      out_shape=jax.ShapeDtypeStruct((1, 1), jnp.float32),
      out_specs=pl.BlockSpec(memory_space=pltpu.SMEM),
  )(pt)
  return ece.reshape(1)
